# baseline (speedup 1.0000x reference)
import functools

import jax
import jax.numpy as jnp
from jax import lax
from jax.experimental import pallas as pl
from jax.experimental.pallas import tpu as pltpu
from jax.experimental.pallas import tpu_sc as plsc

B, L, D = 4, 4096, 1024
L_SC = 1280
L_TC = L - L_SC
BLOCK_L = 256
NC, NS = 2, 16
NW = NC * NS
ROWS_W = B * L_SC // NW
W_PER_B = L_SC // ROWS_W
R = 16
NCHUNK = ROWS_W // R


def _sc_add(x_hbm, pos_hbm, out_hbm, xbuf, pbuf, obuf, xs0, xs1, ps0, ps1,
            os0, os1):
    xsem = (xs0, xs1)
    psem = (ps0, ps1)
    osem = (os0, os1)
    wid = lax.axis_index("s") * NC + lax.axis_index("c")
    bi = wid // W_PER_B
    lbase = (wid % W_PER_B) * ROWS_W

    def start_in(cc, b):
        l0 = lbase + cc * R
        pltpu.async_copy(x_hbm.at[bi, pl.ds(l0, R), :], xbuf.at[b], xsem[b])
        pltpu.async_copy(pos_hbm.at[pl.ds(l0, R), :], pbuf.at[b], psem[b])

    def wait_in(cc, b):
        l0 = lbase + cc * R
        pltpu.make_async_copy(x_hbm.at[bi, pl.ds(l0, R), :], xbuf.at[b],
                              xsem[b]).wait()
        pltpu.make_async_copy(pos_hbm.at[pl.ds(l0, R), :], pbuf.at[b],
                              psem[b]).wait()

    def wait_out(cc, b):
        l0 = lbase + cc * R
        pltpu.make_async_copy(obuf.at[b], out_hbm.at[bi, pl.ds(l0, R), :],
                              osem[b]).wait()

    start_in(0, 0)
    start_in(1, 1)

    @pl.loop(0, NCHUNK, step=2)
    def _(c):
        for b in range(2):
            cc = c + b
            wait_in(cc, b)

            @pl.when(cc >= 2)
            def _():
                wait_out(cc - 2, b)

            @plsc.parallel_loop(0, R * D // 16, unroll=8)
            def _(j):
                r = j >> 6
                col = (j & (D // 16 - 1)) * 16
                s = pl.ds(col, 16)
                obuf[b, r, s] = xbuf[b, r, s] + pbuf[b, r, s]

            l0 = lbase + cc * R
            pltpu.async_copy(obuf.at[b], out_hbm.at[bi, pl.ds(l0, R), :],
                             osem[b])

            @pl.when(cc + 2 < NCHUNK)
            def _():
                start_in(cc + 2, b)

    wait_out(NCHUNK - 2, 0)
    wait_out(NCHUNK - 1, 1)


_sc_kernel = functools.partial(
    pl.kernel,
    mesh=plsc.VectorSubcoreMesh(core_axis_name="c", subcore_axis_name="s"),
    out_type=jax.ShapeDtypeStruct((B, L_SC, D), jnp.float32),
    scratch_types=[
        pltpu.VMEM((2, R, D), jnp.float32),
        pltpu.VMEM((2, R, D), jnp.float32),
        pltpu.VMEM((2, R, D), jnp.float32),
        pltpu.SemaphoreType.DMA,
        pltpu.SemaphoreType.DMA,
        pltpu.SemaphoreType.DMA,
        pltpu.SemaphoreType.DMA,
        pltpu.SemaphoreType.DMA,
        pltpu.SemaphoreType.DMA,
    ],
)(_sc_add)


def _tc_add(x_ref, pos_ref, o_ref):
    o_ref[...] = x_ref[...] + pos_ref[...][None, :, :]


def _tc_kernel(x, pos_table):
    off = L_SC // BLOCK_L
    return pl.pallas_call(
        _tc_add,
        grid=(L_TC // BLOCK_L,),
        in_specs=[
            pl.BlockSpec((B, BLOCK_L, D), lambda l: (0, l + off, 0)),
            pl.BlockSpec((BLOCK_L, D), lambda l: (l + off, 0)),
        ],
        out_specs=pl.BlockSpec((B, BLOCK_L, D), lambda l: (0, l + off, 0)),
        out_shape=jax.ShapeDtypeStruct((B, L, D), x.dtype),
    )(x, pos_table)


def kernel(x, pos_table):
    sc_out = _sc_kernel(x, pos_table)
    tc_out = _tc_kernel(x, pos_table)
    return lax.dynamic_update_slice(tc_out, sc_out, (0, 0, 0))


# speedup vs baseline: 1.1404x; 1.1404x over previous
import functools

import jax
import jax.numpy as jnp
from jax import lax
from jax.experimental import pallas as pl
from jax.experimental.pallas import tpu as pltpu
from jax.experimental.pallas import tpu_sc as plsc

B, L, D = 4, 4096, 1024
NC, NS = 2, 16
NW = NC * NS
LROWS_W = L // NW
R = 16
NPC = LROWS_W // R
NK = NPC * B


def _sc_add(x_hbm, pos_hbm, out_hbm, xbuf, pbuf, obuf, xs0, xs1, ps0, ps1,
            os0, os1):
    xsem = (xs0, xs1)
    psem = (ps0, ps1)
    osem = (os0, os1)
    wid = lax.axis_index("s") * NC + lax.axis_index("c")
    lbase = wid * LROWS_W

    def start_x(k, slot):
        bi = k & (B - 1)
        l0 = lbase + (k >> 2) * R
        pltpu.async_copy(x_hbm.at[bi, pl.ds(l0, R), :], xbuf.at[slot],
                         xsem[slot])

    def wait_x(k, slot):
        bi = k & (B - 1)
        l0 = lbase + (k >> 2) * R
        pltpu.make_async_copy(x_hbm.at[bi, pl.ds(l0, R), :], xbuf.at[slot],
                              xsem[slot]).wait()

    def start_pos(pc, slot):
        l0 = lbase + pc * R
        pltpu.async_copy(pos_hbm.at[pl.ds(l0, R), :], pbuf.at[slot],
                         psem[slot])

    def wait_pos(pc, slot):
        l0 = lbase + pc * R
        pltpu.make_async_copy(pos_hbm.at[pl.ds(l0, R), :], pbuf.at[slot],
                              psem[slot]).wait()

    def start_out(k, slot):
        bi = k & (B - 1)
        l0 = lbase + (k >> 2) * R
        pltpu.async_copy(obuf.at[slot], out_hbm.at[bi, pl.ds(l0, R), :],
                         osem[slot])

    def wait_out(k, slot):
        bi = k & (B - 1)
        l0 = lbase + (k >> 2) * R
        pltpu.make_async_copy(obuf.at[slot], out_hbm.at[bi, pl.ds(l0, R), :],
                              osem[slot]).wait()

    start_pos(0, 0)
    start_x(0, 0)
    start_x(1, 1)

    @pl.loop(0, NPC, step=2)
    def _(pc0):
        for pp in range(2):
            pc = pc0 + pp
            wait_pos(pc, pp)

            @pl.when(pc + 1 < NPC)
            def _():
                start_pos(pc + 1, 1 - pp)

            for b in range(B):
                k = pc * B + b
                slot = b & 1
                wait_x(k, slot)

                @pl.when(k >= 2)
                def _():
                    wait_out(k - 2, slot)

                @plsc.parallel_loop(0, R * D // 16, unroll=8)
                def _(j):
                    r = j >> 6
                    col = (j & (D // 16 - 1)) * 16
                    s = pl.ds(col, 16)
                    obuf[slot, r, s] = xbuf[slot, r, s] + pbuf[pp, r, s]

                start_out(k, slot)

                @pl.when(k + 2 < NK)
                def _():
                    start_x(k + 2, slot)

    wait_out(NK - 2, 0)
    wait_out(NK - 1, 1)


_sc_kernel = functools.partial(
    pl.kernel,
    mesh=plsc.VectorSubcoreMesh(core_axis_name="c", subcore_axis_name="s"),
    out_type=jax.ShapeDtypeStruct((B, L, D), jnp.float32),
    scratch_types=[
        pltpu.VMEM((2, R, D), jnp.float32),
        pltpu.VMEM((2, R, D), jnp.float32),
        pltpu.VMEM((2, R, D), jnp.float32),
        pltpu.SemaphoreType.DMA,
        pltpu.SemaphoreType.DMA,
        pltpu.SemaphoreType.DMA,
        pltpu.SemaphoreType.DMA,
        pltpu.SemaphoreType.DMA,
        pltpu.SemaphoreType.DMA,
    ],
)(_sc_add)


def kernel(x, pos_table):
    return _sc_kernel(x, pos_table)


# speedup vs baseline: 1.2127x; 1.0634x over previous
import functools

import jax
import jax.numpy as jnp
from jax import lax
from jax.experimental import pallas as pl
from jax.experimental.pallas import tpu as pltpu
from jax.experimental.pallas import tpu_sc as plsc

B, L, D = 4, 4096, 1024
NC, NS = 2, 16
NW = NC * NS
LROWS_W = L // NW
R = 8
XD = 4
NPC = LROWS_W // R
NK = NPC * B


def _sc_add(x_hbm, pos_hbm, out_hbm, xbuf, pbuf, obuf, xs0, xs1, xs2, xs3,
            ps0, ps1, os0, os1, os2, os3):
    xsem = (xs0, xs1, xs2, xs3)
    psem = (ps0, ps1)
    osem = (os0, os1, os2, os3)
    wid = lax.axis_index("s") * NC + lax.axis_index("c")
    lbase = wid * LROWS_W

    def start_x(k, slot):
        bi = k & (B - 1)
        l0 = lbase + (k >> 2) * R
        pltpu.async_copy(x_hbm.at[bi, pl.ds(l0, R), :], xbuf.at[slot],
                         xsem[slot])

    def wait_x(k, slot):
        bi = k & (B - 1)
        l0 = lbase + (k >> 2) * R
        pltpu.make_async_copy(x_hbm.at[bi, pl.ds(l0, R), :], xbuf.at[slot],
                              xsem[slot]).wait()

    def start_pos(pc, slot):
        l0 = lbase + pc * R
        pltpu.async_copy(pos_hbm.at[pl.ds(l0, R), :], pbuf.at[slot],
                         psem[slot])

    def wait_pos(pc, slot):
        l0 = lbase + pc * R
        pltpu.make_async_copy(pos_hbm.at[pl.ds(l0, R), :], pbuf.at[slot],
                              psem[slot]).wait()

    def start_out(k, slot):
        bi = k & (B - 1)
        l0 = lbase + (k >> 2) * R
        pltpu.async_copy(obuf.at[slot], out_hbm.at[bi, pl.ds(l0, R), :],
                         osem[slot])

    def wait_out(k, slot):
        bi = k & (B - 1)
        l0 = lbase + (k >> 2) * R
        pltpu.make_async_copy(obuf.at[slot], out_hbm.at[bi, pl.ds(l0, R), :],
                              osem[slot]).wait()

    start_pos(0, 0)
    for s in range(XD):
        start_x(s, s)

    @pl.loop(0, NPC, step=2)
    def _(pc0):
        for pp in range(2):
            pc = pc0 + pp
            wait_pos(pc, pp)

            @pl.when(pc + 1 < NPC)
            def _():
                start_pos(pc + 1, 1 - pp)

            for b in range(B):
                k = pc * B + b
                slot = b
                wait_x(k, slot)

                @pl.when(k >= XD)
                def _():
                    wait_out(k - XD, slot)

                @plsc.parallel_loop(0, R * D // 16, unroll=8)
                def _(j):
                    r = j >> 6
                    col = (j & (D // 16 - 1)) * 16
                    s = pl.ds(col, 16)
                    obuf[slot, r, s] = xbuf[slot, r, s] + pbuf[pp, r, s]

                start_out(k, slot)

                @pl.when(k + XD < NK)
                def _():
                    start_x(k + XD, slot)

    for s in range(XD):
        wait_out(NK - XD + s, s)


_sc_kernel = functools.partial(
    pl.kernel,
    mesh=plsc.VectorSubcoreMesh(core_axis_name="c", subcore_axis_name="s"),
    out_type=jax.ShapeDtypeStruct((B, L, D), jnp.float32),
    scratch_types=[
        pltpu.VMEM((XD, R, D), jnp.float32),
        pltpu.VMEM((2, R, D), jnp.float32),
        pltpu.VMEM((XD, R, D), jnp.float32),
        pltpu.SemaphoreType.DMA,
        pltpu.SemaphoreType.DMA,
        pltpu.SemaphoreType.DMA,
        pltpu.SemaphoreType.DMA,
        pltpu.SemaphoreType.DMA,
        pltpu.SemaphoreType.DMA,
        pltpu.SemaphoreType.DMA,
        pltpu.SemaphoreType.DMA,
        pltpu.SemaphoreType.DMA,
        pltpu.SemaphoreType.DMA,
    ],
)(_sc_add)


def kernel(x, pos_table):
    return _sc_kernel(x, pos_table)
